# Initial kernel scaffold; baseline (speedup 1.0000x reference)
import jax
import jax.numpy as jnp
from jax.experimental import pallas as pl


def kernel(x, edge_index, edge_attr, batch, lin0_W, lin0_b, e1_W, e1_b, e2_W, e2_b, root_W, conv_b, gru_Wih, gru_bih, gru_Whh, gru_bhh, lstm_Wih, lstm_bih, lstm_Whh, lstm_bhh, fc1_W, fc1_b, fc2_W, fc2_b):
    raise NotImplementedError("write your pallas kernel here")


# speedup vs baseline: 1.6487x; 1.6487x over previous
import functools

import jax
import jax.numpy as jnp
from jax import lax
from jax.experimental import pallas as pl
from jax.experimental.pallas import tpu as pltpu
from jax.experimental.pallas import tpu_sc as plsc

N = 10000
E = 160000
D = 16
B = 64

NC = 2
NS = 16
NW = NC * NS
CHUNK = 128
E_PAD = 163840
CHUNKS_PER_TILE = E_PAD // (NW * CHUNK)
EDGES_PER_TILE = CHUNKS_PER_TILE * CHUNK
N_PAD = 10240
ROWS_PER_SUB = N_PAD // NS

@functools.cache
def _sc_mesh():
    return plsc.VectorSubcoreMesh(
        core_axis_name="c", subcore_axis_name="s", num_cores=NC,
        num_subcores=NS)


@functools.cache
def _make_sc_gather():
    def body(table_hbm, idx_hbm, out_hbm, idx_v, rows_v, sem):
        c = lax.axis_index("c")
        s = lax.axis_index("s")
        wid = s * NC + c

        pltpu.sync_copy(
            idx_hbm.at[pl.ds(wid * CHUNKS_PER_TILE, CHUNKS_PER_TILE)], idx_v)

        def step(j, carry):
            pltpu.async_copy(
                table_hbm.at[idx_v.at[j]],
                rows_v.at[pl.ds(j * CHUNK, CHUNK)], sem).wait()
            return carry

        lax.fori_loop(0, CHUNKS_PER_TILE, step, 0)
        pltpu.sync_copy(
            rows_v, out_hbm.at[pl.ds(wid * EDGES_PER_TILE, EDGES_PER_TILE)])

    return pl.kernel(
        body,
        out_type=jax.ShapeDtypeStruct((E_PAD, D), jnp.float32),
        mesh=_sc_mesh(),
        scratch_types=[
            pltpu.VMEM((CHUNKS_PER_TILE, CHUNK), jnp.int32),
            pltpu.VMEM((EDGES_PER_TILE, D), jnp.float32),
            pltpu.SemaphoreType.DMA,
        ],
        compiler_params=pltpu.CompilerParams(use_tc_tiling_on_sc=False),
    )


def _sc_gather(table, idx2d):
    return _make_sc_gather()(table, idx2d)


@functools.cache
def _make_sc_scatter():
    def body(vals_hbm, idx_hbm, zeros_hbm, out_hbm, idx_v, rows_v, acc_sh):
        c = lax.axis_index("c")
        s = lax.axis_index("s")
        wid = s * NC + c

        pltpu.sync_copy(zeros_hbm,
                        acc_sh.at[pl.ds(s * ROWS_PER_SUB, ROWS_PER_SUB)])
        pltpu.sync_copy(
            idx_hbm.at[pl.ds(wid * CHUNKS_PER_TILE, CHUNKS_PER_TILE)], idx_v)
        pltpu.sync_copy(
            vals_hbm.at[pl.ds(wid * EDGES_PER_TILE, EDGES_PER_TILE)], rows_v)
        plsc.subcore_barrier()

        def step(j, carry):
            pltpu.sync_copy(rows_v.at[pl.ds(j * CHUNK, CHUNK)],
                            acc_sh.at[idx_v.at[j]], add=True)
            return carry

        lax.fori_loop(0, CHUNKS_PER_TILE, step, 0)
        plsc.subcore_barrier()

        pltpu.sync_copy(
            acc_sh.at[pl.ds(s * ROWS_PER_SUB, ROWS_PER_SUB)],
            out_hbm.at[c, pl.ds(s * ROWS_PER_SUB, ROWS_PER_SUB)],
        )

    return pl.kernel(
        body,
        out_type=jax.ShapeDtypeStruct((NC, N_PAD, D), jnp.float32),
        mesh=_sc_mesh(),
        scratch_types=[
            pltpu.VMEM((CHUNKS_PER_TILE, CHUNK), jnp.int32),
            pltpu.VMEM((EDGES_PER_TILE, D), jnp.float32),
            pltpu.VMEM_SHARED((N_PAD, D), jnp.float32),
        ],
        compiler_params=pltpu.CompilerParams(use_tc_tiling_on_sc=False),
    )


def _sc_scatter(vals, idx2d, zeros_rows):
    return _make_sc_scatter()(vals, idx2d, zeros_rows)


def _pre_body(x_ref, w_ref, b_ref, degp_ref, out_ref, rdeg_ref):
    out_ref[...] = jax.nn.relu(
        jnp.dot(x_ref[...], w_ref[...], preferred_element_type=jnp.float32)
        + b_ref[...]
    )
    deg = degp_ref[0] + degp_ref[1]
    rdeg_ref[...] = 1.0 / jnp.maximum(deg, 1.0)


def _pre(x, lin0_W, lin0_b2, deg_parts):
    blk = 2000
    grid = N // blk
    return pl.pallas_call(
        _pre_body,
        grid=(grid,),
        in_specs=[
            pl.BlockSpec((blk, 128), lambda i: (i, 0)),
            pl.BlockSpec((128, D), lambda i: (0, 0)),
            pl.BlockSpec((1, D), lambda i: (0, 0)),
            pl.BlockSpec((NC, blk, D), lambda i: (0, i, 0)),
        ],
        out_specs=[
            pl.BlockSpec((blk, D), lambda i: (i, 0)),
            pl.BlockSpec((blk, D), lambda i: (i, 0)),
        ],
        out_shape=[
            jax.ShapeDtypeStruct((N, D), jnp.float32),
            jax.ShapeDtypeStruct((N, D), jnp.float32),
        ],
    )(x, lin0_W, lin0_b2, deg_parts)


def _msg_body(ea_ref, sf_ref, w1_ref, b1_ref, w2_ref, b2_ref, t_ref, g_ref,
              msg_ref, *, blk):
    eh = jax.nn.relu(
        jnp.dot(ea_ref[...], w1_ref[...], preferred_element_type=jnp.float32)
        + b1_ref[...]
    )
    we = jnp.dot(eh, w2_ref[...], preferred_element_type=jnp.float32) + b2_ref[...]
    src_t = jnp.dot(sf_ref[...], t_ref[...], preferred_element_type=jnp.float32,
                    precision=lax.Precision.HIGHEST)
    msg = jnp.dot(src_t * we, g_ref[...], preferred_element_type=jnp.float32,
                  precision=lax.Precision.HIGHEST)
    rows = blk * pl.program_id(0) + lax.broadcasted_iota(jnp.int32, (blk, 1), 0)
    msg_ref[...] = jnp.where(rows < E, msg, 0.0)


def _msg(ea_p, src_feat, e1_W, e1_b2, e2_W2, e2_b2, t_mat, g_mat):
    blk = 2048
    grid = E_PAD // blk
    return pl.pallas_call(
        functools.partial(_msg_body, blk=blk),
        grid=(grid,),
        in_specs=[
            pl.BlockSpec((blk, 16), lambda i: (i, 0)),
            pl.BlockSpec((blk, D), lambda i: (i, 0)),
            pl.BlockSpec((16, 128), lambda i: (0, 0)),
            pl.BlockSpec((1, 128), lambda i: (0, 0)),
            pl.BlockSpec((128, 256), lambda i: (0, 0)),
            pl.BlockSpec((1, 256), lambda i: (0, 0)),
            pl.BlockSpec((D, 256), lambda i: (0, 0)),
            pl.BlockSpec((256, D), lambda i: (0, 0)),
        ],
        out_specs=pl.BlockSpec((blk, D), lambda i: (i, 0)),
        out_shape=jax.ShapeDtypeStruct((E_PAD, D), jnp.float32),
    )(ea_p, src_feat, e1_W, e1_b2, e2_W2, e2_b2, t_mat, g_mat)


def _update_body(h_ref, aggp_ref, rdeg_ref, rw_ref, cb_ref, wih_ref, bih_ref,
                 whh_ref, bhh_ref, out_ref):
    h = h_ref[...]
    agg = (aggp_ref[0] + aggp_ref[1]) * rdeg_ref[...]
    m = jax.nn.relu(
        jnp.dot(h, rw_ref[...], preferred_element_type=jnp.float32)
        + agg + cb_ref[...]
    )
    gi = jnp.dot(m, wih_ref[...], preferred_element_type=jnp.float32) + bih_ref[...]
    gh = jnp.dot(h, whh_ref[...], preferred_element_type=jnp.float32) + bhh_ref[...]
    r = jax.nn.sigmoid(gi[:, 0:D] + gh[:, 0:D])
    z = jax.nn.sigmoid(gi[:, D:2 * D] + gh[:, D:2 * D])
    n = jnp.tanh(gi[:, 2 * D:3 * D] + r * gh[:, 2 * D:3 * D])
    out_ref[...] = (1.0 - z) * n + z * h


def _update(h, agg_parts, rdeg, root_W, conv_b2, gru_Wih, gru_bih2, gru_Whh,
            gru_bhh2):
    blk = 2000
    grid = N // blk
    return pl.pallas_call(
        _update_body,
        grid=(grid,),
        in_specs=[
            pl.BlockSpec((blk, D), lambda i: (i, 0)),
            pl.BlockSpec((NC, blk, D), lambda i: (0, i, 0)),
            pl.BlockSpec((blk, D), lambda i: (i, 0)),
            pl.BlockSpec((D, D), lambda i: (0, 0)),
            pl.BlockSpec((1, D), lambda i: (0, 0)),
            pl.BlockSpec((D, 3 * D), lambda i: (0, 0)),
            pl.BlockSpec((1, 3 * D), lambda i: (0, 0)),
            pl.BlockSpec((D, 3 * D), lambda i: (0, 0)),
            pl.BlockSpec((1, 3 * D), lambda i: (0, 0)),
        ],
        out_specs=pl.BlockSpec((blk, D), lambda i: (i, 0)),
        out_shape=jax.ShapeDtypeStruct((N, D), jnp.float32),
    )(h, agg_parts, rdeg, root_W, conv_b2, gru_Wih, gru_bih2, gru_Whh,
      gru_bhh2)


def _s2s_body(out_ref, bcol_ref, brow_ref, wih_ref, bih_ref, whh_ref,
              bhh_ref, fc1w_ref, fc1b_ref, fc2w_ref, fc2b_ref, y_ref):
    nodes = out_ref[...]
    oh = (bcol_ref[...] == lax.broadcasted_iota(jnp.int32, (N, B), 1)
          ).astype(jnp.float32)
    oht = (brow_ref[...] == lax.broadcasted_iota(jnp.int32, (B, N), 0)
           ).astype(jnp.float32)
    q_star = jnp.zeros((B, 2 * D), jnp.float32)
    hs = jnp.zeros((B, D), jnp.float32)
    cs = jnp.zeros((B, D), jnp.float32)
    for _ in range(3):
        gates = (
            jnp.dot(q_star, wih_ref[...], preferred_element_type=jnp.float32)
            + bih_ref[...]
            + jnp.dot(hs, whh_ref[...], preferred_element_type=jnp.float32)
            + bhh_ref[...]
        )
        i_g = jax.nn.sigmoid(gates[:, 0:D])
        f_g = jax.nn.sigmoid(gates[:, D:2 * D])
        g_g = jnp.tanh(gates[:, 2 * D:3 * D])
        o_g = jax.nn.sigmoid(gates[:, 3 * D:4 * D])
        cs = f_g * cs + i_g * g_g
        hs = o_g * jnp.tanh(cs)
        q = hs
        qb = jnp.dot(oh, q, preferred_element_type=jnp.float32)
        e = jnp.sum(nodes * qb, axis=1, keepdims=True)
        a = jnp.exp(e)
        z = jnp.concatenate([a, a * nodes], axis=1)
        u = jnp.dot(oht, z, preferred_element_type=jnp.float32)
        denom = u[:, 0:1]
        r_read = u[:, 1:1 + D] / (denom + 1e-16)
        q_star = jnp.concatenate([q, r_read], axis=1)
    hid = jax.nn.relu(
        jnp.dot(q_star, fc1w_ref[...], preferred_element_type=jnp.float32)
        + fc1b_ref[...]
    )
    y_ref[...] = (
        jnp.dot(hid, fc2w_ref[...], preferred_element_type=jnp.float32)
        + fc2b_ref[...]
    )


def _s2s(out, bcol, brow, lstm_Wih, lstm_bih2, lstm_Whh, lstm_bhh2, fc1_W,
         fc1_b2, fc2_W, fc2_b2):
    return pl.pallas_call(
        _s2s_body,
        out_shape=jax.ShapeDtypeStruct((B, 1), jnp.float32),
    )(out, bcol, brow, lstm_Wih, lstm_bih2, lstm_Whh, lstm_bhh2, fc1_W,
      fc1_b2, fc2_W, fc2_b2)


def kernel(x, edge_index, edge_attr, batch, lin0_W, lin0_b, e1_W, e1_b,
           e2_W, e2_b, root_W, conv_b, gru_Wih, gru_bih, gru_Whh, gru_bhh,
           lstm_Wih, lstm_bih, lstm_Whh, lstm_bhh, fc1_W, fc1_b, fc2_W,
           fc2_b):
    pad = E_PAD - E
    src = jnp.concatenate([edge_index[0], jnp.zeros((pad,), jnp.int32)])
    dst = jnp.concatenate([edge_index[1], jnp.zeros((pad,), jnp.int32)])
    src2d = src.reshape(E_PAD // CHUNK, CHUNK)
    dst2d = dst.reshape(E_PAD // CHUNK, CHUNK)
    ea_p = jnp.concatenate(
        [edge_attr, jnp.zeros((pad, edge_attr.shape[1]), jnp.float32)])
    ones_p = jnp.concatenate(
        [jnp.ones((E, D), jnp.float32), jnp.zeros((pad, D), jnp.float32)])
    zeros_rows = jnp.zeros((ROWS_PER_SUB, D), jnp.float32)

    e2_W2 = e2_W.reshape(128, D, D).transpose(0, 2, 1).reshape(128, D * D)
    e2_b2 = e2_b.reshape(D, D).T.reshape(1, D * D)
    eye = jnp.eye(D, dtype=jnp.float32)
    t_mat = jnp.tile(eye, (1, D))
    g_mat = jnp.repeat(eye, D, axis=0)

    deg_parts = _sc_scatter(ones_p, dst2d, zeros_rows)
    h, rdeg = _pre(x, lin0_W, lin0_b.reshape(1, D), deg_parts)

    for _ in range(3):
        src_feat = _sc_gather(h, src2d)
        msg = _msg(ea_p, src_feat, e1_W, e1_b.reshape(1, 128), e2_W2, e2_b2,
                   t_mat, g_mat)
        agg_parts = _sc_scatter(msg, dst2d, zeros_rows)
        h = _update(h, agg_parts, rdeg, root_W, conv_b.reshape(1, D),
                    gru_Wih, gru_bih.reshape(1, 3 * D), gru_Whh,
                    gru_bhh.reshape(1, 3 * D))

    return _s2s(h, batch.reshape(N, 1), batch.reshape(1, N), lstm_Wih,
                lstm_bih.reshape(1, 4 * D), lstm_Whh,
                lstm_bhh.reshape(1, 4 * D), fc1_W, fc1_b.reshape(1, 128),
                fc2_W, fc2_b.reshape(1, 1))
